# Initial kernel scaffold; baseline (speedup 1.0000x reference)
import jax
import jax.numpy as jnp
from jax.experimental import pallas as pl


def kernel(x, edge_index, batch, W1, b1, W2, b2):
    raise NotImplementedError("write your pallas kernel here")


# speedup vs baseline: 17.6049x; 17.6049x over previous
import functools

import jax
import jax.numpy as jnp
from jax import lax
from jax.experimental import pallas as pl
from jax.experimental.pallas import tpu as pltpu
from jax.experimental.pallas import tpu_sc as plsc

N = 10000
E = 320000
IN_CH = 128
OUT_CH = 64
HID = 512
G = 64

NC = 2
NS = 16
NW = NC * NS
EB = 128

NP = 10240
RPT = NP // NS
NCHUNK = (E + NW * EB - 1) // (NW * EB)
EPW = NCHUNK * EB
EP = EPW * NW

_mesh = plsc.VectorSubcoreMesh(core_axis_name="c", subcore_axis_name="s")


@functools.partial(
    pl.kernel,
    mesh=_mesh,
    out_type=jax.ShapeDtypeStruct((NW, NP), jnp.float32),
    compiler_params=pltpu.CompilerParams(needs_layout_passes=False),
    scratch_types=[
        pltpu.VMEM((NP,), jnp.float32),
        pltpu.VMEM((EB,), jnp.int32),
    ],
)
def _deg_kernel(dst_hbm, z_hbm, out_hbm, deg_v, idx_v):
    c = lax.axis_index("c")
    s = lax.axis_index("s")
    wid = s * NC + c
    pltpu.sync_copy(z_hbm, deg_v)
    ones16 = jnp.full((16,), 1.0, dtype=jnp.float32)

    def body(i, _):
        pltpu.sync_copy(dst_hbm.at[pl.ds(wid * EPW + i * EB, EB)], idx_v)
        for j in range(EB // 16):
            idx = idx_v[pl.ds(j * 16, 16)]
            plsc.addupdate_scatter(deg_v, [idx], ones16)
        return 0

    lax.fori_loop(0, NCHUNK, body, 0)
    pltpu.sync_copy(deg_v, out_hbm.at[wid])


def _make_scatter(C):
    @functools.partial(
        pl.kernel,
        mesh=_mesh,
        out_type=jax.ShapeDtypeStruct((NC, NP, C), jnp.float32),
        compiler_params=pltpu.CompilerParams(use_tc_tiling_on_sc=False),
        scratch_types=[
            pltpu.VMEM((EB,), jnp.int32),
            pltpu.VMEM((EB,), jnp.int32),
            pltpu.VMEM((EB, C), jnp.float32),
            pltpu.VMEM_SHARED((NP, C), jnp.float32),
            pltpu.SemaphoreType.DMA,
        ],
    )
    def scatter_kernel(xp_hbm, src_hbm, dst_hbm, z_hbm, out_hbm,
                       si, di, rows, acc, sem):
        c = lax.axis_index("c")
        s = lax.axis_index("s")
        wid = s * NC + c
        pltpu.sync_copy(z_hbm, acc.at[pl.ds(s * RPT, RPT)])
        plsc.subcore_barrier()

        def body(i, _):
            base = wid * EPW + i * EB
            pltpu.sync_copy(src_hbm.at[pl.ds(base, EB)], si)
            pltpu.sync_copy(dst_hbm.at[pl.ds(base, EB)], di)
            pltpu.async_copy(xp_hbm.at[si], rows, sem).wait()
            pltpu.sync_copy(rows, acc.at[di], add=True)
            return 0

        lax.fori_loop(0, NCHUNK, body, 0)
        plsc.subcore_barrier()
        pltpu.sync_copy(acc.at[pl.ds(s * RPT, RPT)],
                        out_hbm.at[c, pl.ds(s * RPT, RPT)])

    return scatter_kernel


_scatter128 = _make_scatter(IN_CH)
_scatter64 = _make_scatter(OUT_CH)


def _norm_body(cnt_ref, x_ref, dis_ref, xp_ref):
    tot = jnp.sum(cnt_ref[...], axis=1, keepdims=True) + 1.0
    dis = lax.rsqrt(tot)
    dis_ref[...] = dis
    xp_ref[...] = x_ref[...] * dis


def _mid_body(dis_ref, xp_ref, s1_ref, w1_ref, b1_ref, w2_ref, mp_ref):
    dis = dis_ref[...]
    t = (s1_ref[0] + s1_ref[1] + xp_ref[...]) * dis
    h1 = jnp.maximum(
        jnp.dot(t, w1_ref[...], preferred_element_type=jnp.float32)
        + b1_ref[...], 0.0)
    m = jnp.dot(h1, w2_ref[...], preferred_element_type=jnp.float32)
    mp_ref[...] = m * dis


def _tail_body(dis_ref, mp_ref, s2_ref, b2_ref, batch_ref, out_ref,
               sacc, cacc):
    i = pl.program_id(0)

    @pl.when(i == 0)
    def _():
        sacc[...] = jnp.zeros_like(sacc)
        cacc[...] = jnp.zeros_like(cacc)

    h2 = jnp.maximum(
        (s2_ref[0] + s2_ref[1] + mp_ref[...]) * dis_ref[...] + b2_ref[...],
        0.0)
    blk = batch_ref.shape[0]
    p = (batch_ref[...] == lax.broadcasted_iota(jnp.int32, (blk, G), 1)
         ).astype(jnp.float32)
    dn = (((0,), (0,)), ((), ()))
    sacc[...] += lax.dot_general(p, h2, dn,
                                 preferred_element_type=jnp.float32)
    ones = jnp.ones((blk, 1), dtype=jnp.float32)
    cacc[...] += lax.dot_general(p, ones, dn,
                                 preferred_element_type=jnp.float32)

    @pl.when(i == pl.num_programs(0) - 1)
    def _():
        out_ref[...] = sacc[...] / jnp.maximum(cacc[...], 1.0)


def kernel(x, edge_index, batch, W1, b1, W2, b2):
    f32 = jnp.float32
    src = jnp.concatenate(
        [edge_index[0], jnp.full((EP - E,), N, dtype=jnp.int32)])
    dst = jnp.concatenate(
        [edge_index[1], jnp.full((EP - E,), N, dtype=jnp.int32)])
    x_pad = jnp.zeros((NP, IN_CH), f32).at[:N].set(x)
    batch_pad = jnp.full((NP, 1), G, jnp.int32).at[:N, 0].set(batch)

    z_np = jnp.zeros((NP,), f32)
    z128 = jnp.zeros((RPT, IN_CH), f32)
    z64 = jnp.zeros((RPT, OUT_CH), f32)

    cnts = _deg_kernel(dst, z_np)
    cnts_t = cnts.T

    BLK = NP // 8
    grid = (8,)
    dis, xp = pl.pallas_call(
        _norm_body,
        grid=grid,
        in_specs=[
            pl.BlockSpec((BLK, NW), lambda i: (i, 0)),
            pl.BlockSpec((BLK, IN_CH), lambda i: (i, 0)),
        ],
        out_specs=[
            pl.BlockSpec((BLK, 1), lambda i: (i, 0)),
            pl.BlockSpec((BLK, IN_CH), lambda i: (i, 0)),
        ],
        out_shape=[
            jax.ShapeDtypeStruct((NP, 1), f32),
            jax.ShapeDtypeStruct((NP, IN_CH), f32),
        ],
    )(cnts_t, x_pad)

    s1 = _scatter128(xp, src, dst, z128)

    mp = pl.pallas_call(
        _mid_body,
        grid=grid,
        in_specs=[
            pl.BlockSpec((BLK, 1), lambda i: (i, 0)),
            pl.BlockSpec((BLK, IN_CH), lambda i: (i, 0)),
            pl.BlockSpec((NC, BLK, IN_CH), lambda i: (0, i, 0)),
            pl.BlockSpec((IN_CH, HID), lambda i: (0, 0)),
            pl.BlockSpec((1, HID), lambda i: (0, 0)),
            pl.BlockSpec((HID, OUT_CH), lambda i: (0, 0)),
        ],
        out_specs=pl.BlockSpec((BLK, OUT_CH), lambda i: (i, 0)),
        out_shape=jax.ShapeDtypeStruct((NP, OUT_CH), f32),
    )(dis, xp, s1, W1, b1.reshape(1, HID), W2)

    s2 = _scatter64(mp, src, dst, z64)

    out = pl.pallas_call(
        _tail_body,
        grid=grid,
        in_specs=[
            pl.BlockSpec((BLK, 1), lambda i: (i, 0)),
            pl.BlockSpec((BLK, OUT_CH), lambda i: (i, 0)),
            pl.BlockSpec((NC, BLK, OUT_CH), lambda i: (0, i, 0)),
            pl.BlockSpec((1, OUT_CH), lambda i: (0, 0)),
            pl.BlockSpec((BLK, 1), lambda i: (i, 0)),
        ],
        out_specs=pl.BlockSpec((G, OUT_CH), lambda i: (0, 0)),
        out_shape=jax.ShapeDtypeStruct((G, OUT_CH), f32),
        scratch_shapes=[
            pltpu.VMEM((G, OUT_CH), f32),
            pltpu.VMEM((G, 1), f32),
        ],
    )(dis, mp, s2, b2.reshape(1, OUT_CH), batch_pad)

    return out
